# baseline (speedup 1.0000x reference)
import functools

import jax
import jax.numpy as jnp
from jax.experimental import pallas as pl

_TX = 512


def _chamfer_kernel(x_ref, y_ref, out_ref, *, n, m, d):
    yv = y_ref[0]
    y2 = jnp.sum(yv * yv, axis=1, keepdims=True).T

    def body(i, carry):
        row_sum, col_min = carry
        xs = x_ref[0, pl.ds(i * _TX, _TX), :]
        x2 = jnp.sum(xs * xs, axis=1, keepdims=True)
        xy = jax.lax.dot_general(
            xs, yv, (((1,), (1,)), ((), ())),
            preferred_element_type=jnp.float32)
        dist = x2 + y2 - 2.0 * xy
        row_sum = row_sum + jnp.sum(jnp.min(dist, axis=1))
        col_min = jnp.minimum(col_min, jnp.min(dist, axis=0, keepdims=True))
        return row_sum, col_min

    init = (jnp.float32(0.0), jnp.full((1, m), jnp.inf, jnp.float32))
    row_sum, col_min = jax.lax.fori_loop(0, n // _TX, body, init)
    dist_xy = row_sum / n
    dist_yx = jnp.sum(col_min) / m
    out_ref[0] = jnp.maximum(dist_xy, dist_yx).reshape(1, 1)


def kernel(x, y):
    b, n, d = x.shape
    m = y.shape[1]
    dp = 8
    xp = jnp.pad(x, ((0, 0), (0, 0), (0, dp - d)))
    yp = jnp.pad(y, ((0, 0), (0, 0), (0, dp - d)))

    per_item = pl.pallas_call(
        functools.partial(_chamfer_kernel, n=n, m=m, d=dp),
        grid=(b,),
        in_specs=[
            pl.BlockSpec((1, n, dp), lambda i: (i, 0, 0)),
            pl.BlockSpec((1, m, dp), lambda i: (i, 0, 0)),
        ],
        out_specs=pl.BlockSpec((1, 1, 1), lambda i: (i, 0, 0)),
        out_shape=jax.ShapeDtypeStruct((b, 1, 1), jnp.float32),
    )(xp, yp)
    return jnp.mean(per_item)


# speedup vs baseline: 1.6039x; 1.6039x over previous
import functools

import jax
import jax.numpy as jnp
from jax.experimental import pallas as pl


_TX = 512


def _chamfer_kernel(xa_ref, yt_ref, out_ref, *, n, m):
    yt = yt_ref[0]

    def body(i, carry):
        row_sum, col_min = carry
        xs = xa_ref[0, pl.ds(i * _TX, _TX), :]
        dist = jax.lax.dot_general(
            xs, yt, (((1,), (0,)), ((), ())),
            preferred_element_type=jnp.float32)
        row_sum = row_sum + jnp.sum(jnp.min(dist, axis=1))
        col_min = jnp.minimum(col_min, jnp.min(dist, axis=0, keepdims=True))
        return row_sum, col_min

    init = (jnp.float32(0.0), jnp.full((1, m), jnp.inf, jnp.float32))
    row_sum, col_min = jax.lax.fori_loop(0, n // _TX, body, init)
    dist_xy = row_sum / n
    dist_yx = jnp.sum(col_min) / m
    out_ref[0] = jnp.maximum(dist_xy, dist_yx).reshape(1, 1)


def kernel(x, y):
    b, n, _ = x.shape
    m = y.shape[1]
    zeros = jnp.zeros((b, n, 1), jnp.float32)
    ones = jnp.ones((b, n, 1), jnp.float32)
    x2 = jnp.sum(x * x, axis=2, keepdims=True)
    y2 = jnp.sum(y * y, axis=2, keepdims=True)
    xa = jnp.concatenate([-2.0 * x, x2, ones, zeros, zeros, zeros], axis=2)
    ya = jnp.concatenate([y, ones, y2, zeros, zeros, zeros], axis=2)
    yt = jnp.swapaxes(ya, 1, 2)

    per_item = pl.pallas_call(
        functools.partial(_chamfer_kernel, n=n, m=m),
        grid=(b,),
        in_specs=[
            pl.BlockSpec((1, n, 8), lambda i: (i, 0, 0)),
            pl.BlockSpec((1, 8, m), lambda i: (i, 0, 0)),
        ],
        out_specs=pl.BlockSpec((1, 1, 1), lambda i: (i, 0, 0)),
        out_shape=jax.ShapeDtypeStruct((b, 1, 1), jnp.float32),
    )(xa, yt)
    return jnp.mean(per_item)
